# baseline (speedup 1.0000x reference)
import functools

import jax
import jax.numpy as jnp
from jax import lax
from jax.experimental import pallas as pl
from jax.experimental.pallas import tpu as pltpu
from jax.experimental.pallas import tpu_sc as plsc

N = 10000
D = 128
NC = 2
NS = 16
L = 16
NW = NC * NS
K = 32
DUMP = N
NACC = 10112
ROWS_PER_SUB = NACC // NS


def _sc_aggregate(x, row, col, n_chunks):
    mesh = plsc.VectorSubcoreMesh(core_axis_name="c", subcore_axis_name="s")

    @functools.partial(
        pl.kernel,
        mesh=mesh,
        compiler_params=pltpu.CompilerParams(needs_layout_passes=False),
        out_type=[
            jax.ShapeDtypeStruct((NC, NACC, D), jnp.float32),
            jax.ShapeDtypeStruct((NW, NACC), jnp.float32),
        ],
        scratch_types=[
            [pltpu.VMEM((K,), jnp.int32)] * 6,
            [pltpu.VMEM((K,), jnp.int32)] * 6,
            [pltpu.VMEM((K, D), jnp.float32)] * 3,
            pltpu.VMEM((NACC,), jnp.float32),
            pltpu.VMEM_SHARED((NACC, D), jnp.float32),
            [pltpu.SemaphoreType.DMA] * 3,
            [pltpu.SemaphoreType.DMA] * 6,
        ],
    )
    def sc_kernel(x_hbm, row_hbm, col_hbm, acc_out, cnt_out,
                  row_vs, col_vs, rows_vs, cnt_v, acc_sh, gsem, isem):
        cid = lax.axis_index("c")
        sid = lax.axis_index("s")
        wid = sid * NC + cid
        base_e = wid * (n_chunks * K)
        lane = lax.iota(jnp.int32, L)

        def idx_load(t, c):
            off = base_e + c * K
            pltpu.async_copy(row_hbm.at[pl.ds(off, K)], row_vs[t], isem[t])
            pltpu.async_copy(col_hbm.at[pl.ds(off, K)], col_vs[t], isem[t])

        def idx_wait(t):
            pltpu.make_async_copy(
                row_hbm.at[pl.ds(0, K)], row_vs[t], isem[t]).wait()
            pltpu.make_async_copy(
                col_hbm.at[pl.ds(0, K)], col_vs[t], isem[t]).wait()

        def prep(t):
            for j in range(K // L):
                sl = pl.ds(j * L, L)
                rv = row_vs[t][sl]
                cv = col_vs[t][sl]
                valid = rv != cv
                counts, last = plsc.scan_count(cv, mask=valid)
                plsc.addupdate_scatter(
                    cnt_v, [cv], counts.astype(jnp.float32), mask=last)
                col_vs[t][sl] = jnp.where(valid, cv, DUMP + lane)

        def gather_issue(t, p):
            pltpu.async_copy(x_hbm.at[row_vs[t]], rows_vs[p], gsem[p])

        def gather_wait(t, p):
            pltpu.make_async_copy(
                x_hbm.at[row_vs[t]], rows_vs[p], gsem[p]).wait()

        for t in range(6):
            idx_load(t, t)

        @pl.loop(0, K)
        def _(r):
            for j in range(D // L):
                rows_vs[0][r, pl.ds(j * L, L)] = jnp.zeros((L,), jnp.float32)

        @pl.loop(0, NACC // L)
        def _(i):
            cnt_v[pl.ds(i * L, L)] = jnp.zeros((L,), jnp.float32)

        base_r = sid * ROWS_PER_SUB
        for b in range(ROWS_PER_SUB // K):
            pltpu.sync_copy(rows_vs[0], acc_sh.at[pl.ds(base_r + b * K, K)])
        rem = ROWS_PER_SUB % K
        if rem:
            pltpu.sync_copy(
                rows_vs[0].at[pl.ds(0, rem)],
                acc_sh.at[pl.ds(base_r + ROWS_PER_SUB - rem, rem)])
        plsc.subcore_barrier()

        for t in range(3):
            idx_wait(t)
            prep(t)
            gather_issue(t, t)

        @pl.loop(0, n_chunks // 6)
        def _(i):
            for u in range(6):
                p = u % 3
                q = (u + 3) % 6
                c = i * 6 + u
                gather_wait(u, p)
                pltpu.sync_copy(rows_vs[p], acc_sh.at[col_vs[u]], add=True)

                @pl.when(c + 3 < n_chunks)
                def _():
                    idx_wait(q)
                    prep(q)
                    gather_issue(q, p)

                @pl.when(c + 6 < n_chunks)
                def _():
                    idx_load(u, c + 6)

        plsc.subcore_barrier()
        pltpu.sync_copy(acc_sh.at[pl.ds(base_r, ROWS_PER_SUB)],
                        acc_out.at[cid, pl.ds(base_r, ROWS_PER_SUB)])
        pltpu.sync_copy(cnt_v, cnt_out.at[wid])

    return sc_kernel(x, row, col)


def _tc_finish(x, acc, cnt, ones_w, w1t, w2t, b, g, be):
    def body(x_ref, acc_ref, cnt_ref, ones_ref, w1_ref, w2_ref,
             b_ref, g_ref, be_ref, o_ref):
        xv = x_ref[...]
        a = acc_ref[0, :N] + acc_ref[1, :N]
        agg = a + xv
        call = lax.dot_general(
            cnt_ref[...], ones_ref[...],
            dimension_numbers=(((0,), (0,)), ((), ())),
            preferred_element_type=jnp.float32)
        cnt = call[:N] + 1.0
        aggr = agg / jnp.maximum(cnt, 1.0)
        h = (jnp.dot(xv, w1_ref[...], preferred_element_type=jnp.float32)
             + jnp.dot(aggr, w2_ref[...], preferred_element_type=jnp.float32)
             + b_ref[...])
        nrm = jnp.sqrt(jnp.sum(h * h, axis=1, keepdims=True))
        h = h / jnp.maximum(nrm, 1e-12)
        mu = jnp.mean(h, axis=0, keepdims=True)
        var = jnp.mean((h - mu) ** 2, axis=0, keepdims=True)
        h = (h - mu) * lax.rsqrt(var + 1e-5) * g_ref[...] + be_ref[...]
        o_ref[...] = jnp.maximum(h, 0.0) + xv

    return pl.pallas_call(
        body,
        out_shape=jax.ShapeDtypeStruct((N, D), jnp.float32),
    )(x, acc, cnt, ones_w, w1t, w2t, b, g, be)


def kernel(x, edge_index, edge_w, pos_w1, pos_w2, lin_w, lin_b,
           bn_gamma, bn_beta):
    row = edge_index[0]
    col = edge_index[1]
    e = row.shape[0]
    per_w = 6 * K * (-(-e // (NW * 6 * K)))
    n_chunks = per_w // K
    pad = n_chunks * NW * K - e
    if pad:
        row = jnp.concatenate([row, jnp.zeros((pad,), row.dtype)])
        col = jnp.concatenate([col, jnp.zeros((pad,), col.dtype)])
    acc, cnt = _sc_aggregate(x, row, col, n_chunks)
    ones_w = jnp.ones((NW, 1), jnp.float32)
    w1t = lin_w[:, :D].T
    w2t = lin_w[:, D:].T
    return _tc_finish(x, acc, cnt, ones_w, w1t, w2t,
                      lin_b[None], bn_gamma[None], bn_beta[None])


# speedup vs baseline: 1.1833x; 1.1833x over previous
import functools

import jax
import jax.numpy as jnp
from jax import lax
from jax.experimental import pallas as pl
from jax.experimental.pallas import tpu as pltpu
from jax.experimental.pallas import tpu_sc as plsc

N = 10000
D = 128
NC = 2
NS = 16
L = 16
NW = NC * NS
K = 32
DUMP = N
NACC = 10112
ROWS_PER_SUB = NACC // NS


def _sc_aggregate(x, row, col, n_chunks):
    mesh = plsc.VectorSubcoreMesh(core_axis_name="c", subcore_axis_name="s")

    @functools.partial(
        pl.kernel,
        mesh=mesh,
        compiler_params=pltpu.CompilerParams(needs_layout_passes=False),
        out_type=[
            jax.ShapeDtypeStruct((NC, NACC, D), jnp.float32),
            jax.ShapeDtypeStruct((NW, NACC), jnp.float32),
        ],
        scratch_types=[
            [pltpu.VMEM((K,), jnp.int32)] * 4,
            [pltpu.VMEM((K,), jnp.int32)] * 4,
            [pltpu.VMEM((K, D), jnp.float32)] * 2,
            pltpu.VMEM((NACC,), jnp.float32),
            pltpu.VMEM_SHARED((NACC, D), jnp.float32),
            [pltpu.SemaphoreType.DMA] * 2,
            [pltpu.SemaphoreType.DMA] * 4,
        ],
    )
    def sc_kernel(x_hbm, row_hbm, col_hbm, acc_out, cnt_out,
                  row_vs, col_vs, rows_vs, cnt_v, acc_sh, gsem, isem):
        cid = lax.axis_index("c")
        sid = lax.axis_index("s")
        wid = sid * NC + cid
        base_e = wid * (n_chunks * K)
        lane = lax.iota(jnp.int32, L)

        def idx_load(t, c):
            off = base_e + c * K
            pltpu.async_copy(row_hbm.at[pl.ds(off, K)], row_vs[t], isem[t])
            pltpu.async_copy(col_hbm.at[pl.ds(off, K)], col_vs[t], isem[t])

        def idx_wait(t):
            pltpu.make_async_copy(
                row_hbm.at[pl.ds(0, K)], row_vs[t], isem[t]).wait()
            pltpu.make_async_copy(
                col_hbm.at[pl.ds(0, K)], col_vs[t], isem[t]).wait()

        def prep(t):
            for j in range(K // L):
                sl = pl.ds(j * L, L)
                rv = row_vs[t][sl]
                cv = col_vs[t][sl]
                valid = rv != cv
                counts, last = plsc.scan_count(cv, mask=valid)
                plsc.addupdate_scatter(
                    cnt_v, [cv], counts.astype(jnp.float32), mask=last)
                col_vs[t][sl] = jnp.where(valid, cv, DUMP + lane)

        def gather_issue(t, p):
            pltpu.async_copy(x_hbm.at[row_vs[t]], rows_vs[p], gsem[p])

        def gather_wait(t, p):
            pltpu.make_async_copy(
                x_hbm.at[row_vs[t]], rows_vs[p], gsem[p]).wait()

        for t in range(4):
            idx_load(t, t)

        @pl.loop(0, K)
        def _(r):
            for j in range(D // L):
                rows_vs[0][r, pl.ds(j * L, L)] = jnp.zeros((L,), jnp.float32)

        @pl.loop(0, NACC // L)
        def _(i):
            cnt_v[pl.ds(i * L, L)] = jnp.zeros((L,), jnp.float32)

        base_r = sid * ROWS_PER_SUB
        for b in range(ROWS_PER_SUB // K):
            pltpu.sync_copy(rows_vs[0], acc_sh.at[pl.ds(base_r + b * K, K)])
        rem = ROWS_PER_SUB % K
        if rem:
            pltpu.sync_copy(
                rows_vs[0].at[pl.ds(0, rem)],
                acc_sh.at[pl.ds(base_r + ROWS_PER_SUB - rem, rem)])
        plsc.subcore_barrier()

        for t in range(2):
            idx_wait(t)
            prep(t)
            gather_issue(t, t)

        @pl.loop(0, n_chunks // 4)
        def _(i):
            for u in range(4):
                p = u % 2
                q = (u + 2) % 4
                c = i * 4 + u
                gather_wait(u, p)
                pltpu.sync_copy(rows_vs[p], acc_sh.at[col_vs[u]], add=True)

                @pl.when(c + 2 < n_chunks)
                def _():
                    idx_wait(q)
                    prep(q)
                    gather_issue(q, p)

                @pl.when(c + 4 < n_chunks)
                def _():
                    idx_load(u, c + 4)

        plsc.subcore_barrier()
        pltpu.sync_copy(acc_sh.at[pl.ds(base_r, ROWS_PER_SUB)],
                        acc_out.at[cid, pl.ds(base_r, ROWS_PER_SUB)])
        pltpu.sync_copy(cnt_v, cnt_out.at[wid])

    return sc_kernel(x, row, col)


def _tc_finish(x, acc, cnt, ones_w, w1t, w2t, b, g, be):
    def body(x_ref, acc_ref, cnt_ref, ones_ref, w1_ref, w2_ref,
             b_ref, g_ref, be_ref, o_ref):
        xv = x_ref[...]
        a = acc_ref[0, :N] + acc_ref[1, :N]
        agg = a + xv
        call = lax.dot_general(
            cnt_ref[...], ones_ref[...],
            dimension_numbers=(((0,), (0,)), ((), ())),
            preferred_element_type=jnp.float32)
        cnt = call[:N] + 1.0
        aggr = agg / jnp.maximum(cnt, 1.0)
        h = (jnp.dot(xv, w1_ref[...], preferred_element_type=jnp.float32)
             + jnp.dot(aggr, w2_ref[...], preferred_element_type=jnp.float32)
             + b_ref[...])
        nrm = jnp.sqrt(jnp.sum(h * h, axis=1, keepdims=True))
        h = h / jnp.maximum(nrm, 1e-12)
        mu = jnp.mean(h, axis=0, keepdims=True)
        var = jnp.mean((h - mu) ** 2, axis=0, keepdims=True)
        h = (h - mu) * lax.rsqrt(var + 1e-5) * g_ref[...] + be_ref[...]
        o_ref[...] = jnp.maximum(h, 0.0) + xv

    return pl.pallas_call(
        body,
        out_shape=jax.ShapeDtypeStruct((N, D), jnp.float32),
    )(x, acc, cnt, ones_w, w1t, w2t, b, g, be)


def kernel(x, edge_index, edge_w, pos_w1, pos_w2, lin_w, lin_b,
           bn_gamma, bn_beta):
    row = edge_index[0]
    col = edge_index[1]
    e = row.shape[0]
    per_w = 4 * K * (-(-e // (NW * 4 * K)))
    n_chunks = per_w // K
    pad = n_chunks * NW * K - e
    if pad:
        row = jnp.concatenate([row, jnp.zeros((pad,), row.dtype)])
        col = jnp.concatenate([col, jnp.zeros((pad,), col.dtype)])
    acc, cnt = _sc_aggregate(x, row, col, n_chunks)
    ones_w = jnp.ones((NW, 1), jnp.float32)
    w1t = lin_w[:, :D].T
    w2t = lin_w[:, D:].T
    return _tc_finish(x, acc, cnt, ones_w, w1t, w2t,
                      lin_b[None], bn_gamma[None], bn_beta[None])


# speedup vs baseline: 1.8280x; 1.5448x over previous
import functools

import jax
import jax.numpy as jnp
from jax import lax
from jax.experimental import pallas as pl
from jax.experimental.pallas import tpu as pltpu
from jax.experimental.pallas import tpu_sc as plsc

N = 10000
D = 128
NC = 2
NS = 16
L = 16
NW = NC * NS
K = 32
DUMP = N
NACC = 10112
ROWS_PER_SUB = NACC // NS


def _sc_aggregate(x, row, col, n_chunks):
    mesh = plsc.VectorSubcoreMesh(core_axis_name="c", subcore_axis_name="s")

    @functools.partial(
        pl.kernel,
        mesh=mesh,
        compiler_params=pltpu.CompilerParams(needs_layout_passes=False),
        out_type=[
            jax.ShapeDtypeStruct((NC, NACC, D), jnp.float32),
            jax.ShapeDtypeStruct((NW, NACC), jnp.float32),
        ],
        scratch_types=[
            [pltpu.VMEM((K,), jnp.int32)] * 4,
            [pltpu.VMEM((K,), jnp.int32)] * 4,
            [pltpu.VMEM((K, D), jnp.float32)] * 2,
            pltpu.VMEM((NACC,), jnp.float32),
            pltpu.VMEM_SHARED((NACC, D), jnp.float32),
            [pltpu.SemaphoreType.DMA] * 2,
            [pltpu.SemaphoreType.DMA] * 4,
        ],
    )
    def sc_kernel(x_hbm, row_hbm, col_hbm, acc_out, cnt_out,
                  row_vs, col_vs, rows_vs, cnt_v, acc_sh, gsem, isem):
        cid = lax.axis_index("c")
        sid = lax.axis_index("s")
        wid = sid * NC + cid
        base_e = wid * (n_chunks * K)
        lane = lax.iota(jnp.int32, L)

        def idx_load(t, c):
            off = base_e + c * K
            pltpu.async_copy(row_hbm.at[pl.ds(off, K)], row_vs[t], isem[t])
            pltpu.async_copy(col_hbm.at[pl.ds(off, K)], col_vs[t], isem[t])

        def idx_wait(t):
            pltpu.make_async_copy(
                row_hbm.at[pl.ds(0, K)], row_vs[t], isem[t]).wait()
            pltpu.make_async_copy(
                col_hbm.at[pl.ds(0, K)], col_vs[t], isem[t]).wait()

        def prep(t):
            for j in range(K // L):
                sl = pl.ds(j * L, L)
                rv = row_vs[t][sl]
                cv = col_vs[t][sl]
                valid = rv != cv
                counts, last = plsc.scan_count(cv, mask=valid)
                plsc.addupdate_scatter(
                    cnt_v, [cv], counts.astype(jnp.float32), mask=last)
                col_vs[t][sl] = jnp.where(valid, cv, DUMP + lane)

        def gather_issue(t, p):
            pltpu.async_copy(x_hbm.at[row_vs[t]], rows_vs[p], gsem[p])

        def gather_wait(t, p):
            pltpu.make_async_copy(
                x_hbm.at[row_vs[t]], rows_vs[p], gsem[p]).wait()

        for t in range(4):
            idx_load(t, t)

        @pl.loop(0, K)
        def _(r):
            for j in range(D // L):
                rows_vs[0][r, pl.ds(j * L, L)] = jnp.zeros((L,), jnp.float32)

        @pl.loop(0, NACC // L)
        def _(i):
            cnt_v[pl.ds(i * L, L)] = jnp.zeros((L,), jnp.float32)

        base_r = sid * ROWS_PER_SUB
        for b in range(ROWS_PER_SUB // K):
            pltpu.sync_copy(rows_vs[0], acc_sh.at[pl.ds(base_r + b * K, K)])
        rem = ROWS_PER_SUB % K
        if rem:
            pltpu.sync_copy(
                rows_vs[0].at[pl.ds(0, rem)],
                acc_sh.at[pl.ds(base_r + ROWS_PER_SUB - rem, rem)])
        plsc.subcore_barrier()

        for t in range(2):
            idx_wait(t)
            prep(t)
            gather_issue(t, t)

        @pl.loop(0, n_chunks // 4)
        def _(i):
            for u in range(4):
                p = u % 2
                q = (u + 2) % 4
                c = i * 4 + u
                gather_wait(u, p)
                pltpu.sync_copy(rows_vs[p], acc_sh.at[col_vs[u]], add=True)

                @pl.when(c + 2 < n_chunks)
                def _():
                    idx_wait(q)
                    prep(q)
                    gather_issue(q, p)

                @pl.when(c + 4 < n_chunks)
                def _():
                    idx_load(u, c + 4)

        plsc.subcore_barrier()
        pltpu.sync_copy(acc_sh.at[pl.ds(base_r, ROWS_PER_SUB)],
                        acc_out.at[cid, pl.ds(base_r, ROWS_PER_SUB)])
        pltpu.sync_copy(cnt_v, cnt_out.at[wid])

    return sc_kernel(x, row, col)


def _tc_finish(x, acc, cnt, ones_w, w1t, w2t, b, g, be):
    def body(x_ref, acc_ref, cnt_ref, ones_ref, w1_ref, w2_ref,
             b_ref, g_ref, be_ref, o_ref):
        xv = x_ref[...]
        a = acc_ref[0, :N] + acc_ref[1, :N]
        agg = a + xv
        call = lax.dot_general(
            cnt_ref[...], ones_ref[...],
            dimension_numbers=(((0,), (0,)), ((), ())),
            preferred_element_type=jnp.float32)
        cnt = call[:N] + 1.0
        aggr = agg / jnp.maximum(cnt, 1.0)
        h = (jnp.dot(xv, w1_ref[...], preferred_element_type=jnp.float32)
             + jnp.dot(aggr, w2_ref[...], preferred_element_type=jnp.float32)
             + b_ref[...])
        nrm = jnp.sqrt(jnp.sum(h * h, axis=1, keepdims=True))
        h = h / jnp.maximum(nrm, 1e-12)
        mu = jnp.mean(h, axis=0, keepdims=True)
        var = jnp.mean((h - mu) ** 2, axis=0, keepdims=True)
        h = (h - mu) * lax.rsqrt(var + 1e-5) * g_ref[...] + be_ref[...]
        o_ref[...] = jnp.maximum(h, 0.0) + xv

    return pl.pallas_call(
        body,
        out_shape=jax.ShapeDtypeStruct((N, D), jnp.float32),
    )(x, acc, cnt, ones_w, w1t, w2t, b, g, be)


def kernel(x, edge_index, edge_w, pos_w1, pos_w2, lin_w, lin_b,
           bn_gamma, bn_beta):
    row = edge_index[0]
    col = edge_index[1]
    e = row.shape[0]
    per_w = 4 * K * (-(-e // (NW * 4 * K)))
    n_chunks = per_w // K
    pad = n_chunks * NW * K - e
    if pad:
        pad_idx = jnp.arange(pad, dtype=row.dtype) % N
        row = jnp.concatenate([row, pad_idx])
        col = jnp.concatenate([col, pad_idx])
    acc, cnt = _sc_aggregate(x, row, col, n_chunks)
    ones_w = jnp.ones((NW, 1), jnp.float32)
    w1t = lin_w[:, :D].T
    w2t = lin_w[:, D:].T
    return _tc_finish(x, acc, cnt, ones_w, w1t, w2t,
                      lin_b[None], bn_gamma[None], bn_beta[None])
